# baseline (speedup 1.0000x reference)
import jax
import jax.numpy as jnp
from jax.experimental import pallas as pl

N_TOK = 8192
D_IN = 2048
D_HID = 16384
TOPK = 32

TM_A = 1024
TH_A = 512


def _enc_body(x_ref, w_ref, b_ref, o_ref):
    xh = x_ref[...].astype(jnp.bfloat16)
    wh = w_ref[...].astype(jnp.bfloat16)
    dims = (((1,), (1,)), ((), ()))
    acc = jax.lax.dot_general(xh, wh, dims, preferred_element_type=jnp.float32)
    o_ref[...] = acc + b_ref[...]


def _encode(x, W_enc, b_enc):
    return pl.pallas_call(
        _enc_body,
        grid=(N_TOK // TM_A, D_HID // TH_A),
        in_specs=[
            pl.BlockSpec((TM_A, D_IN), lambda m, h: (m, 0)),
            pl.BlockSpec((TH_A, D_IN), lambda m, h: (h, 0)),
            pl.BlockSpec((1, TH_A), lambda m, h: (0, h)),
        ],
        out_specs=pl.BlockSpec((TM_A, TH_A), lambda m, h: (m, h)),
        out_shape=jax.ShapeDtypeStruct((N_TOK, D_HID), jnp.float32),
    )(x, W_enc, b_enc.reshape(1, D_HID))


TM_B = 128


def _count_ge(bits, mid):
    return jnp.sum((bits >= mid).astype(jnp.int32), axis=1, keepdims=True)


def _sel_body(h_ref, o_ref):
    pos = jnp.maximum(h_ref[...], 0.0)
    bits = jax.lax.bitcast_convert_type(pos, jnp.int32)

    cm = jnp.max(bits.reshape(TM_B, 128, 128), axis=1)

    def cstep(_, carry):
        lo, hi = carry
        mid = (lo + hi) >> 1
        ge = jnp.sum((cm >= mid).astype(jnp.int32), axis=1, keepdims=True) >= TOPK
        return jnp.where(ge, mid, lo), jnp.where(ge, hi, mid)

    clo0 = jnp.zeros((TM_B, 1), jnp.int32)
    chi0 = jnp.full((TM_B, 1), 0x7F800000, jnp.int32)
    m32, _ = jax.lax.fori_loop(0, 31, cstep, (clo0, chi0))
    m1 = jnp.max(cm, axis=1, keepdims=True)

    def step(_, carry):
        lo, hi = carry
        mid = (lo + hi) >> 1
        ge = _count_ge(bits, mid) >= TOPK
        return jnp.where(ge, mid, lo), jnp.where(ge, hi, mid)

    lo, hi = jax.lax.fori_loop(0, 26, step, (m32, m1 + 1))

    def wcond(carry):
        l, h = carry
        return jnp.any((h - l) > 1)

    lo, _ = jax.lax.while_loop(wcond, lambda c: step(0, c), (lo, hi))
    o_ref[...] = jnp.where(bits >= lo, pos, 0.0)


def _select(h_pre):
    return pl.pallas_call(
        _sel_body,
        grid=(N_TOK // TM_B,),
        in_specs=[pl.BlockSpec((TM_B, D_HID), lambda m: (m, 0))],
        out_specs=pl.BlockSpec((TM_B, D_HID), lambda m: (m, 0)),
        out_shape=jax.ShapeDtypeStruct((N_TOK, D_HID), jnp.float32),
    )(h_pre)


TM_C = 1024
TH_C = 2048


def _dec_body(h_ref, w_ref, b_ref, o_ref):
    j = pl.program_id(1)

    @pl.when(j == 0)
    def _():
        o_ref[...] = jnp.broadcast_to(b_ref[...], o_ref.shape)

    h = h_ref[...].astype(jnp.bfloat16)
    o_ref[...] += jax.lax.dot_general(
        h, w_ref[...], (((1,), (0,)), ((), ())),
        preferred_element_type=jnp.float32,
    )


def _decode(h_sparse, W_dec_t_bf16, b_dec):
    return pl.pallas_call(
        _dec_body,
        grid=(N_TOK // TM_C, D_HID // TH_C),
        in_specs=[
            pl.BlockSpec((TM_C, TH_C), lambda m, h: (m, h)),
            pl.BlockSpec((TH_C, D_IN), lambda m, h: (h, 0)),
            pl.BlockSpec((1, D_IN), lambda m, h: (0, 0)),
        ],
        out_specs=pl.BlockSpec((TM_C, D_IN), lambda m, h: (m, 0)),
        out_shape=jax.ShapeDtypeStruct((N_TOK, D_IN), jnp.float32),
    )(h_sparse, W_dec_t_bf16, b_dec.reshape(1, D_IN))


def kernel(x, W_enc, b_enc, W_dec, b_dec):
    h_pre = _encode(x, W_enc, b_enc)
    h_sparse = _select(h_pre)
    w_dec_t = W_dec.T.astype(jnp.bfloat16)
    recon = _decode(h_sparse, w_dec_t, b_dec)
    return (recon, h_sparse, h_pre)


# speedup vs baseline: 1.0630x; 1.0630x over previous
import jax
import jax.numpy as jnp
from jax.experimental import pallas as pl

N_TOK = 8192
D_IN = 2048
D_HID = 16384
TOPK = 32

TM_A = 1024
TH_A = 512


def _enc_body(x_ref, w_ref, b_ref, o_ref):
    xh = x_ref[...].astype(jnp.bfloat16)
    wh = w_ref[...].astype(jnp.bfloat16)
    dims = (((1,), (1,)), ((), ()))
    acc = jax.lax.dot_general(xh, wh, dims, preferred_element_type=jnp.float32)
    o_ref[...] = acc + b_ref[...]


def _encode(x, W_enc, b_enc):
    return pl.pallas_call(
        _enc_body,
        grid=(N_TOK // TM_A, D_HID // TH_A),
        in_specs=[
            pl.BlockSpec((TM_A, D_IN), lambda m, h: (m, 0)),
            pl.BlockSpec((TH_A, D_IN), lambda m, h: (h, 0)),
            pl.BlockSpec((1, TH_A), lambda m, h: (0, h)),
        ],
        out_specs=pl.BlockSpec((TM_A, TH_A), lambda m, h: (m, h)),
        out_shape=jax.ShapeDtypeStruct((N_TOK, D_HID), jnp.float32),
    )(x, W_enc, b_enc.reshape(1, D_HID))


TM_B = 128


def _count_ge(bits, mid):
    return jnp.sum((bits >= mid).astype(jnp.int32), axis=1, keepdims=True)


def _sel_body(h_ref, o_ref):
    pos = jnp.maximum(h_ref[...], 0.0)
    bits = jax.lax.bitcast_convert_type(pos, jnp.int32)

    def step(_, carry):
        lo, hi = carry
        mid = (lo + hi) >> 1
        ge = _count_ge(bits, mid) >= TOPK
        return jnp.where(ge, mid, lo), jnp.where(ge, hi, mid)

    lo0 = jnp.zeros((TM_B, 1), jnp.int32)
    hi0 = jnp.full((TM_B, 1), 0x7F800000, jnp.int32)
    lo, _ = jax.lax.fori_loop(0, 31, step, (lo0, hi0))
    o_ref[...] = jnp.where(bits >= lo, pos, 0.0)


def _select(h_pre):
    return pl.pallas_call(
        _sel_body,
        grid=(N_TOK // TM_B,),
        in_specs=[pl.BlockSpec((TM_B, D_HID), lambda m: (m, 0))],
        out_specs=pl.BlockSpec((TM_B, D_HID), lambda m: (m, 0)),
        out_shape=jax.ShapeDtypeStruct((N_TOK, D_HID), jnp.float32),
    )(h_pre)


TM_C = 1024
TH_C = 2048


def _dec_body(h_ref, w_ref, b_ref, o_ref):
    j = pl.program_id(1)

    @pl.when(j == 0)
    def _():
        o_ref[...] = jnp.broadcast_to(b_ref[...], o_ref.shape)

    h = h_ref[...].astype(jnp.bfloat16)
    o_ref[...] += jax.lax.dot_general(
        h, w_ref[...], (((1,), (0,)), ((), ())),
        preferred_element_type=jnp.float32,
    )


def _decode(h_sparse, W_dec_t_bf16, b_dec):
    return pl.pallas_call(
        _dec_body,
        grid=(N_TOK // TM_C, D_HID // TH_C),
        in_specs=[
            pl.BlockSpec((TM_C, TH_C), lambda m, h: (m, h)),
            pl.BlockSpec((TH_C, D_IN), lambda m, h: (h, 0)),
            pl.BlockSpec((1, D_IN), lambda m, h: (0, 0)),
        ],
        out_specs=pl.BlockSpec((TM_C, D_IN), lambda m, h: (m, 0)),
        out_shape=jax.ShapeDtypeStruct((N_TOK, D_IN), jnp.float32),
    )(h_sparse, W_dec_t_bf16, b_dec.reshape(1, D_IN))


def kernel(x, W_enc, b_enc, W_dec, b_dec):
    h_pre = _encode(x, W_enc, b_enc)
    h_sparse = _select(h_pre)
    w_dec_t = W_dec.T.astype(jnp.bfloat16)
    recon = _decode(h_sparse, w_dec_t, b_dec)
    return (recon, h_sparse, h_pre)
